# baseline (speedup 1.0000x reference)
import functools

import jax
import jax.numpy as jnp
from jax import lax
from jax.experimental import pallas as pl
from jax.experimental.pallas import tpu as pltpu
from jax.experimental.pallas import tpu_sc as plsc

_N = 26
_K = 100000
_B = 16384
_NC = 2
_LANES = 16
_CH = 4096
_NCH = _B // _CH
_CK = 25088
_NBK = -(-_K // _CK)
_FB = 4096


def _sc_gather_body(p_hbm, idx_hbm, out_hbm, row_v, i0, i1, i2, i3, g0, g1,
                    sem_row, sem_idx, sem_o0, sem_o1):
    wid = lax.axis_index("s") * _NC + lax.axis_index("c")

    @pl.when(wid < _N)
    def _():
        idx_bufs = (i0, i1, i2, i3)
        g_bufs = (g0, g1)
        osems = (sem_o0, sem_o1)

        row_cp = pltpu.async_copy(p_hbm.at[wid], row_v, sem_row)
        icps = [
            pltpu.async_copy(
                idx_hbm.at[wid, pl.ds(c * _CH, _CH)], idx_bufs[c], sem_idx)
            for c in range(_NCH)
        ]
        row_cp.wait()

        ocps = [None, None]
        for c in range(_NCH):
            b = c % 2
            icps[c].wait()
            if ocps[b] is not None:
                ocps[b].wait()

            def _gather(ib, gb):
                @plsc.parallel_loop(0, _CH, _LANES, unroll=8)
                def _g(i):
                    sl = pl.ds(i, _LANES)
                    gb[sl] = plsc.load_gather(row_v, [ib[sl]])

            _gather(idx_bufs[c], g_bufs[b])
            ocps[b] = pltpu.async_copy(
                g_bufs[b], out_hbm.at[wid, pl.ds(c * _CH, _CH)], osems[b])
        ocps[0].wait()
        ocps[1].wait()


@functools.lru_cache(maxsize=None)
def _sc_gather():
  return functools.partial(
    pl.kernel,
    out_type=jax.ShapeDtypeStruct((_N, _B), jnp.float32),
    mesh=plsc.VectorSubcoreMesh(core_axis_name="c", subcore_axis_name="s"),
    compiler_params=pltpu.CompilerParams(
        needs_layout_passes=False, skip_device_barrier=True),
    scratch_types=[
        pltpu.VMEM((_K,), jnp.float32),
        pltpu.VMEM((_CH,), jnp.int32),
        pltpu.VMEM((_CH,), jnp.int32),
        pltpu.VMEM((_CH,), jnp.int32),
        pltpu.VMEM((_CH,), jnp.int32),
        pltpu.VMEM((_CH,), jnp.float32),
        pltpu.VMEM((_CH,), jnp.float32),
        pltpu.SemaphoreType.DMA,
        pltpu.SemaphoreType.DMA,
        pltpu.SemaphoreType.DMA,
        pltpu.SemaphoreType.DMA,
    ],
  )(_sc_gather_body)


def _tc_sum_body(p_ref, s_ref):
    i = pl.program_id(0)

    @pl.when(i == 0)
    def _():
        s_ref[...] = jnp.zeros_like(s_ref)

    x = p_ref[...]
    col = i * _CK + lax.broadcasted_iota(jnp.int32, x.shape, 1)
    sig = jnp.where(col < _K, jax.nn.sigmoid(x), 0.0)
    part = jnp.sum(sig, axis=1, keepdims=True)
    s_ref[...] += jnp.broadcast_to(part, s_ref.shape)


def _tc_finish_body(g_ref, s_ref, o_ref):
    s = s_ref[:, 0:1]
    o_ref[...] = jnp.log(jax.nn.sigmoid(g_ref[...]) / s)


def kernel(data, p_aux):
    idx_t = data.T
    s = pl.pallas_call(
        _tc_sum_body,
        grid=(_NBK,),
        in_specs=[pl.BlockSpec((_N, _CK), lambda i: (0, i))],
        out_specs=pl.BlockSpec((_N, 128), lambda i: (0, 0)),
        out_shape=jax.ShapeDtypeStruct((_N, 128), jnp.float32),
    )(p_aux)
    g_t = _sc_gather()(p_aux, idx_t)
    out_t = pl.pallas_call(
        _tc_finish_body,
        grid=(_B // _FB,),
        in_specs=[
            pl.BlockSpec((_N, _FB), lambda i: (0, i)),
            pl.BlockSpec((_N, 128), lambda i: (0, 0)),
        ],
        out_specs=pl.BlockSpec((_N, _FB), lambda i: (0, i)),
        out_shape=jax.ShapeDtypeStruct((_N, _B), jnp.float32),
    )(g_t, s)
    return out_t.T


# speedup vs baseline: 1.0175x; 1.0175x over previous
import functools

import jax
import jax.numpy as jnp
from jax import lax
from jax.experimental import pallas as pl
from jax.experimental.pallas import tpu as pltpu
from jax.experimental.pallas import tpu_sc as plsc

_N = 26
_K = 100000
_B = 16384
_NC = 2
_LANES = 16
_CH = 4096
_NCH = _B // _CH
_CK = 25088
_NBK = -(-_K // _CK)
_FB = 4096


def _sc_gather_body(p_hbm, idx_hbm, out_hbm, row_v, i0, i1, i2, i3, g0, g1,
                    sem_row, sem_idx, sem_o0, sem_o1):
    wid = lax.axis_index("s") * _NC + lax.axis_index("c")

    @pl.when(wid < _N)
    def _():
        idx_bufs = (i0, i1, i2, i3)
        g_bufs = (g0, g1)
        osems = (sem_o0, sem_o1)

        row_cp = pltpu.async_copy(p_hbm.at[wid], row_v, sem_row)
        icps = [
            pltpu.async_copy(
                idx_hbm.at[wid, pl.ds(c * _CH, _CH)], idx_bufs[c], sem_idx)
            for c in range(_NCH)
        ]
        row_cp.wait()

        ocps = [None, None]
        for c in range(_NCH):
            b = c % 2
            icps[c].wait()
            if ocps[b] is not None:
                ocps[b].wait()

            def _gather(ib, gb):
                @plsc.parallel_loop(0, _CH, _LANES, unroll=8)
                def _g(i):
                    sl = pl.ds(i, _LANES)
                    gb[sl] = plsc.load_gather(row_v, [ib[sl]])

            _gather(idx_bufs[c], g_bufs[b])
            ocps[b] = pltpu.async_copy(
                g_bufs[b], out_hbm.at[wid, pl.ds(c * _CH, _CH)], osems[b])
        ocps[0].wait()
        ocps[1].wait()


@functools.lru_cache(maxsize=None)
def _sc_gather():
  return functools.partial(
    pl.kernel,
    out_type=jax.ShapeDtypeStruct((_N, _B), jnp.float32),
    mesh=plsc.VectorSubcoreMesh(core_axis_name="c", subcore_axis_name="s"),
    compiler_params=pltpu.CompilerParams(needs_layout_passes=False),
    scratch_types=[
        pltpu.VMEM((_K,), jnp.float32),
        pltpu.VMEM((_CH,), jnp.int32),
        pltpu.VMEM((_CH,), jnp.int32),
        pltpu.VMEM((_CH,), jnp.int32),
        pltpu.VMEM((_CH,), jnp.int32),
        pltpu.VMEM((_CH,), jnp.float32),
        pltpu.VMEM((_CH,), jnp.float32),
        pltpu.SemaphoreType.DMA,
        pltpu.SemaphoreType.DMA,
        pltpu.SemaphoreType.DMA,
        pltpu.SemaphoreType.DMA,
    ],
  )(_sc_gather_body)


def _tc_sum_body(p_ref, s_ref):
    i = pl.program_id(0)

    @pl.when(i == 0)
    def _():
        s_ref[...] = jnp.zeros_like(s_ref)

    x = p_ref[...]
    col = i * _CK + lax.broadcasted_iota(jnp.int32, x.shape, 1)
    sig = jnp.where(col < _K, jax.nn.sigmoid(x), 0.0)
    part = jnp.sum(sig, axis=1, keepdims=True)
    s_ref[...] += jnp.broadcast_to(part, s_ref.shape)


def _tc_finish_body(g_ref, s_ref, o_ref):
    s = s_ref[:, 0:1]
    o_ref[...] = jnp.log(jax.nn.sigmoid(g_ref[...]) / s)


def kernel(data, p_aux):
    idx_t = data.T
    s = pl.pallas_call(
        _tc_sum_body,
        grid=(_NBK,),
        in_specs=[pl.BlockSpec((_N, _CK), lambda i: (0, i))],
        out_specs=pl.BlockSpec((_N, 128), lambda i: (0, 0)),
        out_shape=jax.ShapeDtypeStruct((_N, 128), jnp.float32),
    )(p_aux)
    g_t = _sc_gather()(p_aux, idx_t)
    out_t = pl.pallas_call(
        _tc_finish_body,
        out_shape=jax.ShapeDtypeStruct((_N, _B), jnp.float32),
    )(g_t, s)
    return out_t.T


# speedup vs baseline: 1.0182x; 1.0007x over previous
import functools

import jax
import jax.numpy as jnp
from jax import lax
from jax.experimental import pallas as pl
from jax.experimental.pallas import tpu as pltpu
from jax.experimental.pallas import tpu_sc as plsc

_N = 26
_K = 100000
_B = 16384
_NC = 2
_LANES = 16
_CH = 4096
_NCH = _B // _CH
_CK = 25088
_NBK = -(-_K // _CK)
_FB = 4096


def _sc_gather_body(p_hbm, idx_hbm, out_hbm, row_v, i0, i1, i2, i3, g0, g1,
                    sem_row, sem_idx, sem_o0, sem_o1):
    wid = lax.axis_index("s") * _NC + lax.axis_index("c")

    @pl.when(wid < _N)
    def _():
        idx_bufs = (i0, i1, i2, i3)
        g_bufs = (g0, g1)
        osems = (sem_o0, sem_o1)

        row_cp = pltpu.async_copy(p_hbm.at[wid], row_v, sem_row)
        icps = [
            pltpu.async_copy(
                idx_hbm.at[wid, pl.ds(c * _CH, _CH)], idx_bufs[c], sem_idx)
            for c in range(_NCH)
        ]
        with jax.named_scope("row_wait"):
            row_cp.wait()

        ocps = [None, None]
        with jax.named_scope("gather_all"):
            for c in range(_NCH):
                b = c % 2
                icps[c].wait()
                if ocps[b] is not None:
                    ocps[b].wait()

                def _gather(ib, gb):
                    @plsc.parallel_loop(0, _CH, _LANES, unroll=8)
                    def _g(i):
                        sl = pl.ds(i, _LANES)
                        gb[sl] = plsc.load_gather(row_v, [ib[sl]])

                _gather(idx_bufs[c], g_bufs[b])
                ocps[b] = pltpu.async_copy(
                    g_bufs[b], out_hbm.at[wid, pl.ds(c * _CH, _CH)], osems[b])
        with jax.named_scope("drain"):
            ocps[0].wait()
            ocps[1].wait()


@functools.lru_cache(maxsize=None)
def _sc_gather():
  return functools.partial(
    pl.kernel,
    out_type=jax.ShapeDtypeStruct((_N, _B), jnp.float32),
    mesh=plsc.VectorSubcoreMesh(core_axis_name="c", subcore_axis_name="s"),
    compiler_params=pltpu.CompilerParams(needs_layout_passes=False),
    scratch_types=[
        pltpu.VMEM((_K,), jnp.float32),
        pltpu.VMEM((_CH,), jnp.int32),
        pltpu.VMEM((_CH,), jnp.int32),
        pltpu.VMEM((_CH,), jnp.int32),
        pltpu.VMEM((_CH,), jnp.int32),
        pltpu.VMEM((_CH,), jnp.float32),
        pltpu.VMEM((_CH,), jnp.float32),
        pltpu.SemaphoreType.DMA,
        pltpu.SemaphoreType.DMA,
        pltpu.SemaphoreType.DMA,
        pltpu.SemaphoreType.DMA,
    ],
  )(_sc_gather_body)


def _tc_sum_body(p_ref, s_ref):
    i = pl.program_id(0)

    @pl.when(i == 0)
    def _():
        s_ref[...] = jnp.zeros_like(s_ref)

    x = p_ref[...]
    col = i * _CK + lax.broadcasted_iota(jnp.int32, x.shape, 1)
    sig = jnp.where(col < _K, jax.nn.sigmoid(x), 0.0)
    part = jnp.sum(sig, axis=1, keepdims=True)
    s_ref[...] += jnp.broadcast_to(part, s_ref.shape)


def _tc_finish_body(g_ref, s_ref, o_ref):
    s = s_ref[:, 0:1]
    o_ref[...] = jnp.log(jax.nn.sigmoid(g_ref[...]) / s)


def kernel(data, p_aux):
    idx_t = data.T
    s = pl.pallas_call(
        _tc_sum_body,
        grid=(_NBK,),
        in_specs=[pl.BlockSpec((_N, _CK), lambda i: (0, i))],
        out_specs=pl.BlockSpec((_N, 128), lambda i: (0, 0)),
        out_shape=jax.ShapeDtypeStruct((_N, 128), jnp.float32),
    )(p_aux)
    g_t = _sc_gather()(p_aux, idx_t)
    out_t = pl.pallas_call(
        _tc_finish_body,
        out_shape=jax.ShapeDtypeStruct((_N, _B), jnp.float32),
    )(g_t, s)
    return out_t.T


# speedup vs baseline: 1.0191x; 1.0009x over previous
import functools

import jax
import jax.numpy as jnp
from jax import lax
from jax.experimental import pallas as pl
from jax.experimental.pallas import tpu as pltpu
from jax.experimental.pallas import tpu_sc as plsc

_N = 26
_K = 100000
_B = 16384
_NC = 2
_LANES = 16
_CH = 4096
_NCH = _B // _CH
_CK = 25088
_NBK = -(-_K // _CK)
_FB = 4096


def _sc_gather_body(p_hbm, idx_hbm, out_hbm, row_v, i0, i1, i2, i3, g0, g1,
                    sem_row, sem_idx, sem_o0, sem_o1):
    wid = lax.axis_index("s") * _NC + lax.axis_index("c")

    @pl.when(wid < _N)
    def _():
        idx_bufs = (i0, i1, i2, i3)
        g_bufs = (g0, g1)
        osems = (sem_o0, sem_o1)

        row_cp = pltpu.async_copy(p_hbm.at[wid], row_v, sem_row)
        icps = [
            pltpu.async_copy(
                idx_hbm.at[wid, pl.ds(c * _CH, _CH)], idx_bufs[c], sem_idx)
            for c in range(_NCH)
        ]
        row_cp.wait()

        ocps = [None, None]
        for c in range(_NCH):
            b = c % 2
            icps[c].wait()
            if ocps[b] is not None:
                ocps[b].wait()

            def _gather(ib, gb):
                @plsc.parallel_loop(0, _CH, _LANES, unroll=8)
                def _g(i):
                    sl = pl.ds(i, _LANES)
                    gb[sl] = plsc.load_gather(row_v, [ib[sl]])

            _gather(idx_bufs[c], g_bufs[b])
            ocps[b] = pltpu.async_copy(
                g_bufs[b], out_hbm.at[wid, pl.ds(c * _CH, _CH)], osems[b])
        ocps[0].wait()
        ocps[1].wait()


@functools.lru_cache(maxsize=None)
def _sc_gather():
  return functools.partial(
    pl.kernel,
    out_type=jax.ShapeDtypeStruct((_N, _B), jnp.float32),
    mesh=plsc.VectorSubcoreMesh(core_axis_name="c", subcore_axis_name="s"),
    compiler_params=pltpu.CompilerParams(needs_layout_passes=False),
    scratch_types=[
        pltpu.VMEM((_K,), jnp.float32),
        pltpu.VMEM((_CH,), jnp.int32),
        pltpu.VMEM((_CH,), jnp.int32),
        pltpu.VMEM((_CH,), jnp.int32),
        pltpu.VMEM((_CH,), jnp.int32),
        pltpu.VMEM((_CH,), jnp.float32),
        pltpu.VMEM((_CH,), jnp.float32),
        pltpu.SemaphoreType.DMA,
        pltpu.SemaphoreType.DMA,
        pltpu.SemaphoreType.DMA,
        pltpu.SemaphoreType.DMA,
    ],
  )(_sc_gather_body)


def _tc_sum_body(p_ref, s_ref):
    i = pl.program_id(0)

    @pl.when(i == 0)
    def _():
        s_ref[...] = jnp.zeros_like(s_ref)

    x = p_ref[...]
    col = i * _CK + lax.broadcasted_iota(jnp.int32, x.shape, 1)
    sig = jnp.where(col < _K, jax.nn.sigmoid(x), 0.0)
    part = jnp.sum(sig, axis=1, keepdims=True)
    s_ref[...] += jnp.broadcast_to(part, s_ref.shape)


def _tc_finish_body(g_ref, s_ref, o_ref):
    s = s_ref[:, 0:1]
    o_ref[...] = jnp.log(jax.nn.sigmoid(g_ref[...]) / s)


def kernel(data, p_aux):
    idx_t = data.T
    s = pl.pallas_call(
        _tc_sum_body,
        grid=(_NBK,),
        in_specs=[pl.BlockSpec((_N, _CK), lambda i: (0, i))],
        out_specs=pl.BlockSpec((_N, 128), lambda i: (0, 0)),
        out_shape=jax.ShapeDtypeStruct((_N, 128), jnp.float32),
    )(p_aux)
    g_t = _sc_gather()(p_aux, idx_t)
    out_t = pl.pallas_call(
        _tc_finish_body,
        out_shape=jax.ShapeDtypeStruct((_N, _B), jnp.float32),
    )(g_t, s)
    return out_t.T


# speedup vs baseline: 1.0211x; 1.0020x over previous
import functools

import jax
import jax.numpy as jnp
from jax import lax
from jax.experimental import pallas as pl
from jax.experimental.pallas import tpu as pltpu
from jax.experimental.pallas import tpu_sc as plsc

_N = 26
_K = 100000
_B = 16384
_NC = 2
_LANES = 16
_CH = 4096
_NCH = _B // _CH
_CK = 25088
_NBK = -(-_K // _CK)


def _sc_gather_body(p_hbm, idx_hbm, out_hbm, row_v, i0, i1, i2, i3, g0, g1,
                    sem_row, sem_i0, sem_i1, sem_i2, sem_i3, sem_o0, sem_o1):
    wid = lax.axis_index("s") * _NC + lax.axis_index("c")

    @pl.when(wid < _N)
    def _():
        idx_bufs = (i0, i1, i2, i3)
        g_bufs = (g0, g1)
        isems = (sem_i0, sem_i1, sem_i2, sem_i3)
        osems = (sem_o0, sem_o1)

        row_cp = pltpu.async_copy(p_hbm.at[wid], row_v, sem_row)
        icps = [
            pltpu.async_copy(
                idx_hbm.at[wid, pl.ds(c * _CH, _CH)], idx_bufs[c], isems[c])
            for c in range(_NCH)
        ]
        row_cp.wait()

        ocps = [None, None]
        for c in range(_NCH):
            b = c % 2
            icps[c].wait()
            if ocps[b] is not None:
                ocps[b].wait()

            def _gather(ib, gb):
                @plsc.parallel_loop(0, _CH, _LANES, unroll=8)
                def _g(i):
                    sl = pl.ds(i, _LANES)
                    gb[sl] = plsc.load_gather(row_v, [ib[sl]])

            _gather(idx_bufs[c], g_bufs[b])
            ocps[b] = pltpu.async_copy(
                g_bufs[b], out_hbm.at[wid, pl.ds(c * _CH, _CH)], osems[b])
        ocps[0].wait()
        ocps[1].wait()


@functools.lru_cache(maxsize=None)
def _sc_gather():
  return functools.partial(
    pl.kernel,
    out_type=jax.ShapeDtypeStruct((_N, _B), jnp.float32),
    mesh=plsc.VectorSubcoreMesh(core_axis_name="c", subcore_axis_name="s"),
    compiler_params=pltpu.CompilerParams(needs_layout_passes=False),
    scratch_types=[
        pltpu.VMEM((_K,), jnp.float32),
        pltpu.VMEM((_CH,), jnp.int32),
        pltpu.VMEM((_CH,), jnp.int32),
        pltpu.VMEM((_CH,), jnp.int32),
        pltpu.VMEM((_CH,), jnp.int32),
        pltpu.VMEM((_CH,), jnp.float32),
        pltpu.VMEM((_CH,), jnp.float32),
        pltpu.SemaphoreType.DMA,
        pltpu.SemaphoreType.DMA,
        pltpu.SemaphoreType.DMA,
        pltpu.SemaphoreType.DMA,
        pltpu.SemaphoreType.DMA,
        pltpu.SemaphoreType.DMA,
        pltpu.SemaphoreType.DMA,
    ],
  )(_sc_gather_body)


def _tc_sum_body(p_ref, s_ref):
    i = pl.program_id(0)

    @pl.when(i == 0)
    def _():
        s_ref[...] = jnp.zeros_like(s_ref)

    x = p_ref[...]
    col = i * _CK + lax.broadcasted_iota(jnp.int32, x.shape, 1)
    sig = jnp.where(col < _K, jax.nn.sigmoid(x), 0.0)
    part = jnp.sum(sig, axis=1, keepdims=True)
    s_ref[...] += jnp.broadcast_to(part, s_ref.shape)


def _tc_finish_body(g_ref, s_ref, o_ref):
    s = s_ref[:, 0:1]
    o_ref[...] = jnp.log(jax.nn.sigmoid(g_ref[...]) / s)


def kernel(data, p_aux):
    idx_t = data.T
    s = pl.pallas_call(
        _tc_sum_body,
        grid=(_NBK,),
        in_specs=[pl.BlockSpec((_N, _CK), lambda i: (0, i))],
        out_specs=pl.BlockSpec((_N, 128), lambda i: (0, 0)),
        out_shape=jax.ShapeDtypeStruct((_N, 128), jnp.float32),
    )(p_aux)
    g_t = _sc_gather()(p_aux, idx_t)
    out_t = pl.pallas_call(
        _tc_finish_body,
        out_shape=jax.ShapeDtypeStruct((_N, _B), jnp.float32),
    )(g_t, s)
    return out_t.T
